# Initial kernel scaffold; baseline (speedup 1.0000x reference)
import jax
import jax.numpy as jnp
from jax.experimental import pallas as pl


def kernel(ego_emb, side_emb, rel_emb, q_w, q_b, k_w, k_b, v_w, v_b):
    raise NotImplementedError("write your pallas kernel here")


# speedup vs baseline: 7.2252x; 7.2252x over previous
import functools

import jax
import jax.numpy as jnp
import numpy as np
from jax.experimental import pallas as pl
from jax.experimental.pallas import tpu as pltpu

_TOP_K = 16


def _bdot(a, b):
    return jax.lax.dot_general(a.astype(jnp.bfloat16), b.astype(jnp.bfloat16),
                               (((1,), (1,)), ((), ())),
                               preferred_element_type=jnp.float32)


def _kv_body(side_ref, rel_ref, kw_ref, kb_ref, vw_ref, vb_ref, k_out, v_out):
    side = side_ref[...]
    kin = side * rel_ref[...]
    k_out[...] = _bdot(kin, kw_ref[...]) + kb_ref[...]
    v_out[...] = _bdot(side, vw_ref[...]) + vb_ref[...]


def _main_body(ego_ref, qw_ref, qb_ref, k_ref, v_ref, out_ref, *, scale):
    q = _bdot(ego_ref[...], qw_ref[...]) + qb_ref[...]
    s = _bdot(q, k_ref[...]) / scale
    m = jnp.max(s, axis=1, keepdims=True)
    t = m
    for _ in range(_TOP_K - 1):
        t = jnp.max(jnp.where(s < t, s, -jnp.inf), axis=1, keepdims=True)
    e = jnp.where(s >= t, jnp.exp(s - m), 0.0)
    denom = jnp.sum(e, axis=1, keepdims=True)
    agg = jax.lax.dot_general(e, v_ref[...], (((1,), (0,)), ((), ())),
                              preferred_element_type=jnp.float32,
                              precision=jax.lax.Precision.HIGHEST)
    out_ref[...] = agg / denom


def _build(n_ego, n_side, d, r_block, kv_block):
    scale = np.float32(np.sqrt(d))
    kv_grid = n_side // kv_block
    kv = pl.pallas_call(
        _kv_body,
        grid=(kv_grid,),
        in_specs=[
            pl.BlockSpec((kv_block, d), lambda i: (i, 0)),
            pl.BlockSpec((kv_block, d), lambda i: (i, 0)),
            pl.BlockSpec((d, d), lambda i: (0, 0)),
            pl.BlockSpec((1, d), lambda i: (0, 0)),
            pl.BlockSpec((d, d), lambda i: (0, 0)),
            pl.BlockSpec((1, d), lambda i: (0, 0)),
        ],
        out_specs=[
            pl.BlockSpec((kv_block, d), lambda i: (i, 0)),
            pl.BlockSpec((kv_block, d), lambda i: (i, 0)),
        ],
        out_shape=[
            jax.ShapeDtypeStruct((n_side, d), jnp.float32),
            jax.ShapeDtypeStruct((n_side, d), jnp.float32),
        ],
    )
    main_grid = n_ego // r_block
    main = pl.pallas_call(
        functools.partial(_main_body, scale=scale),
        grid=(main_grid,),
        in_specs=[
            pl.BlockSpec((r_block, d), lambda i: (i, 0)),
            pl.BlockSpec((d, d), lambda i: (0, 0)),
            pl.BlockSpec((1, d), lambda i: (0, 0)),
            pl.BlockSpec((n_side, d), lambda i: (0, 0)),
            pl.BlockSpec((n_side, d), lambda i: (0, 0)),
        ],
        out_specs=pl.BlockSpec((r_block, d), lambda i: (i, 0)),
        out_shape=jax.ShapeDtypeStruct((n_ego, d), jnp.float32),
    )
    return kv, main


def kernel(ego_emb, side_emb, rel_emb, q_w, q_b, k_w, k_b, v_w, v_b):
    n_ego, d = ego_emb.shape
    n_side = side_emb.shape[0]
    r_block = 400 if n_ego % 400 == 0 else n_ego
    kv_block = 2000 if n_side % 2000 == 0 else n_side
    kv, main = _build(n_ego, n_side, d, r_block, kv_block)
    k_mat, v_mat = kv(side_emb, rel_emb, k_w, k_b.reshape(1, d),
                      v_w, v_b.reshape(1, d))
    return main(ego_emb, q_w, q_b.reshape(1, d), k_mat, v_mat)
